# baseline (speedup 1.0000x reference)
import jax
import jax.numpy as jnp
from jax import lax
from jax.experimental import pallas as pl
from jax.experimental.pallas import tpu as pltpu
from jax.experimental.pallas import tpu_sc as plsc

_NC = 2
_NS = 16
_NW = _NC * _NS
_L = 16

_D = 64
_B = 4096 * 200
_CHUNK = 512
_IDXW = 128
_NJ = _CHUNK // _IDXW
_PER_W = _B // _NW
_STEPS = _PER_W // _CHUNK


def _body(ids_hbm, table1_hbm, out1_hbm,
          idx0, idx1, rows0, rows1, gsem0, gsem1, wsem0, wsem1):
    table_hbm = table1_hbm
    out_hbm = out1_hbm
    wid = lax.axis_index("s") * _NC + lax.axis_index("c")
    base = wid * _PER_W

    def stage_idx(c, idxbuf):
        off = pl.multiple_of(base + c * _CHUNK, _CHUNK)
        pltpu.sync_copy(ids_hbm.at[pl.ds(off, _CHUNK)], idxbuf)
        for i in range(_CHUNK // _L):
            sl = pl.ds(i * _L, _L)
            idxbuf[sl] = idxbuf[sl] + 1

    def fire_gathers(idxbuf, rowsbuf, sem):
        for j in range(_NJ):
            sl = pl.ds(j * _IDXW, _IDXW)
            pltpu.async_copy(table_hbm.at[idxbuf.at[sl]], rowsbuf.at[sl], sem)

    def wait_gathers(rowsbuf, sem):
        pltpu.make_async_copy(table_hbm.at[pl.ds(0, _CHUNK)], rowsbuf, sem).wait()

    def fire_wb(c, rowsbuf, sem):
        off = pl.multiple_of(base + c * _CHUNK, _CHUNK)
        pltpu.async_copy(rowsbuf, out_hbm.at[pl.ds(off, _CHUNK)], sem)

    def wait_wb(rowsbuf, sem):
        pltpu.make_async_copy(rowsbuf, out_hbm.at[pl.ds(0, _CHUNK)], sem).wait()

    stage_idx(0, idx0)
    fire_gathers(idx0, rows0, gsem0)

    @pl.loop(0, _STEPS // 2)
    def _pair(k):
        a = 2 * k
        b = a + 1
        stage_idx(b, idx1)

        @pl.when(k > 0)
        def _():
            wait_wb(rows1, wsem1)

        fire_gathers(idx1, rows1, gsem1)
        wait_gathers(rows0, gsem0)
        fire_wb(a, rows0, wsem0)

        @pl.when(k < _STEPS // 2 - 1)
        def _():
            stage_idx(a + 2, idx0)
            wait_wb(rows0, wsem0)
            fire_gathers(idx0, rows0, gsem0)

        @pl.when(k == _STEPS // 2 - 1)
        def _():
            wait_wb(rows0, wsem0)

        wait_gathers(rows1, gsem1)
        fire_wb(b, rows1, wsem1)

    wait_wb(rows1, wsem1)


def kernel(track_ids, embedding):
    b, s = track_ids.shape
    ids = track_ids.astype(jnp.int32).reshape(b * s)
    emb1 = lax.optimization_barrier(embedding.reshape(-1)).reshape(1000001, _D)
    mesh = plsc.VectorSubcoreMesh(core_axis_name="c", subcore_axis_name="s")
    out = pl.kernel(
        _body,
        out_type=jax.ShapeDtypeStruct((_B, _D), jnp.float32),
        mesh=mesh,
        compiler_params=pltpu.CompilerParams(use_tc_tiling_on_sc=False),
        scratch_types=[
            pltpu.VMEM((_CHUNK,), jnp.int32),
            pltpu.VMEM((_CHUNK,), jnp.int32),
            pltpu.VMEM((_CHUNK, _D), jnp.float32),
            pltpu.VMEM((_CHUNK, _D), jnp.float32),
            pltpu.SemaphoreType.DMA,
            pltpu.SemaphoreType.DMA,
            pltpu.SemaphoreType.DMA,
            pltpu.SemaphoreType.DMA,
        ],
    )(ids, emb1)
    return lax.optimization_barrier(out.reshape(-1)).reshape(b, s, _D)


# speedup vs baseline: 1.2181x; 1.2181x over previous
import jax
import jax.numpy as jnp
from jax import lax
from jax.experimental import pallas as pl
from jax.experimental.pallas import tpu as pltpu
from jax.experimental.pallas import tpu_sc as plsc

_NC = 2
_NS = 16
_NW = _NC * _NS
_L = 16

_D = 64
_DP = 128
_V = 1000001
_B = 4096 * 200
_PER_W = _B // _NW
_WAVE = 128
_STAGES = _PER_W // (8 * _WAVE)


def _body(ids_hbm, table_hbm, out_hbm,
          idx_all, g0, g1, c0, c1, gsem0, gsem1, wsem0, wsem1):
    wid = lax.axis_index("s") * _NC + lax.axis_index("c")
    base = wid * _PER_W
    grows = [g0, g1]
    crows = [c0, c1]
    gsem = [gsem0, gsem1]
    wsem = [wsem0, wsem1]

    def fire_gather(s, j, b):
        pltpu.async_copy(table_hbm.at[idx_all.at[s, j]], grows[b], gsem[b])

    def wait_g(b):
        pltpu.make_async_copy(table_hbm.at[pl.ds(0, _WAVE)], grows[b],
                              gsem[b]).wait()

    def compact(b):
        @pl.loop(0, _WAVE)
        def _rows(r):
            for i in range(_D // _L):
                sl = pl.ds(i * _L, _L)
                crows[b][r, sl] = grows[b][r, sl]

    def fire_wb(w, b):
        off = pl.multiple_of(base + w * _WAVE, _WAVE)
        pltpu.async_copy(crows[b], out_hbm.at[pl.ds(off, _WAVE)], wsem[b])

    def wait_wb(b):
        pltpu.make_async_copy(crows[b], out_hbm.at[pl.ds(0, _WAVE)],
                              wsem[b]).wait()

    pltpu.sync_copy(ids_hbm.at[wid], idx_all)

    @pl.loop(0, _STAGES)
    def _shift(s):
        for j in range(8):
            for i in range(_WAVE // _L):
                sl = pl.ds(i * _L, _L)
                idx_all[s, j, sl] = idx_all[s, j, sl] + 1

    fire_gather(0, 0, 0)

    @pl.loop(0, _STAGES)
    def _stage(s):
        for j in range(8):
            b = j % 2
            if j < 7:
                fire_gather(s, j + 1, 1 - b)
            else:
                @pl.when(s + 1 < _STAGES)
                def _():
                    fire_gather(s + 1, 0, 1 - b)
            wait_g(b)
            if j >= 2:
                wait_wb(b)
            else:
                @pl.when(s > 0)
                def _():
                    wait_wb(b)
            compact(b)
            fire_wb(8 * s + j, b)

    wait_wb(0)
    wait_wb(1)


def kernel(track_ids, embedding):
    b, s = track_ids.shape
    ids = track_ids.astype(jnp.int32).reshape(_NW, _STAGES, 8, _WAVE)
    table = jnp.pad(embedding, ((0, 0), (0, _DP - _D)))
    mesh = plsc.VectorSubcoreMesh(core_axis_name="c", subcore_axis_name="s")
    out = pl.kernel(
        _body,
        out_type=jax.ShapeDtypeStruct((_B, _D), jnp.float32),
        mesh=mesh,
        compiler_params=pltpu.CompilerParams(use_tc_tiling_on_sc=True),
        scratch_types=[
            pltpu.VMEM((_STAGES, 8, _WAVE), jnp.int32),
            pltpu.VMEM((_WAVE, _DP), jnp.float32),
            pltpu.VMEM((_WAVE, _DP), jnp.float32),
            pltpu.VMEM((_WAVE, _D), jnp.float32),
            pltpu.VMEM((_WAVE, _D), jnp.float32),
            pltpu.SemaphoreType.DMA,
            pltpu.SemaphoreType.DMA,
            pltpu.SemaphoreType.DMA,
            pltpu.SemaphoreType.DMA,
        ],
    )(ids, table)
    return out.reshape(b, s, _D)
